# baseline (speedup 1.0000x reference)
import functools

import jax
import jax.numpy as jnp
from jax import lax
from jax.experimental import pallas as pl
from jax.experimental.pallas import tpu as pltpu
from jax.experimental.pallas import tpu_sc as plsc

N = 100000
D = 128
B = 256
N_ITERS = 3

L = 16
KD = D // L
NC = 2
NS = 16
NW = NC * NS
SEG_PER_W = B // NW
CHUNK = 256
NEG_BIG = -3.4e38


def _lstm_body(x_ref, h_ref, c_ref, wih_ref, whh_ref, bih_ref, bhh_ref,
               h_out, c_out):
    x = x_ref[...]
    h = h_ref[...]
    gates = lax.dot_general(x, wih_ref[...], (((1,), (1,)), ((), ())),
                            preferred_element_type=jnp.float32)
    gates = gates + lax.dot_general(h, whh_ref[...], (((1,), (1,)), ((), ())),
                                    preferred_element_type=jnp.float32)
    gates = gates + bih_ref[...] + bhh_ref[...]
    i_ = jax.nn.sigmoid(gates[:, 0 * D:1 * D])
    f_ = jax.nn.sigmoid(gates[:, 1 * D:2 * D])
    g_ = jnp.tanh(gates[:, 2 * D:3 * D])
    o_ = jax.nn.sigmoid(gates[:, 3 * D:4 * D])
    c_new = f_ * c_ref[...] + i_ * g_
    h_out[...] = o_ * jnp.tanh(c_new)
    c_out[...] = c_new


def _lstm_step(x, h, c, W_ih, W_hh, b_ih, b_hh):
    return pl.pallas_call(
        _lstm_body,
        out_shape=(jax.ShapeDtypeStruct((B, D), jnp.float32),
                   jax.ShapeDtypeStruct((B, D), jnp.float32)),
    )(x, h, c, W_ih, W_hh, b_ih, b_hh)


def _sc_readout_body(feat_hbm, q_hbm, offs_hbm, out_hbm,
                     off_v, off_s, q_loc, chunks, e_buf,
                     r_buf, m_buf, l_buf, sem0, sem1):
    wid = lax.axis_index("s") * NC + lax.axis_index("c")
    seg0 = wid * SEG_PER_W
    lanes = lax.iota(jnp.int32, 16)
    xor_idx = [jnp.bitwise_xor(lanes, s) for s in (1, 2, 4, 8)]

    def allsum(v):
        for idx in xor_idx:
            v = v + v.at[idx].get(mode="promise_in_bounds")
        return v

    def allmax(v):
        for idx in xor_idx:
            v = jnp.maximum(v, v.at[idx].get(mode="promise_in_bounds"))
        return v

    pltpu.sync_copy(offs_hbm.at[pl.ds(seg0, L)], off_v)
    pltpu.sync_copy(q_hbm.at[pl.ds(seg0, SEG_PER_W)], q_loc)
    offv = off_v[...]
    for j in range(SEG_PER_W + 1):
        off_s[j] = offv[j]

    zero_v = jnp.zeros((L,), jnp.float32)
    for j in range(SEG_PER_W):
        m_buf[j, :] = jnp.full((L,), NEG_BIG, jnp.float32)
        l_buf[j, :] = zero_v
        for k in range(KD):
            r_buf[j, pl.ds(k * L, L)] = zero_v

    w_lo = offv[0]
    w_hi = offv[SEG_PER_W]
    base0 = (w_lo // 8) * 8
    n_chunks = (w_hi - base0 + CHUNK - 1) // CHUNK
    sems = [sem0, sem1]

    def chunk_start(ci):
        return jnp.minimum(base0 + ci * CHUNK, N - CHUNK)

    def issue(ci, slot, sem):
        pltpu.make_async_copy(
            feat_hbm.at[pl.ds(chunk_start(ci), CHUNK)],
            chunks.at[slot], sem).start()

    def wait(slot, sem):
        pltpu.make_async_copy(
            feat_hbm.at[pl.ds(0, CHUNK)], chunks.at[slot], sem).wait()

    def process_span(j, buf, ra, rb):
        q_vecs = [q_loc[j, pl.ds(k * L, L)] for k in range(KD)]
        g_lo = ra // L
        g_hi = (rb + L - 1) // L

        def grp_a(gi, mv):
            base = gi * L
            ev = zero_v
            for p in range(L):
                prods = [buf[base + p, pl.ds(k * L, L)] * q_vecs[k]
                         for k in range(KD)]
                while len(prods) > 1:
                    prods = [prods[i] + prods[i + 1]
                             for i in range(0, len(prods), 2)]
                ev = jnp.where(lanes == p, allsum(prods[0]), ev)
            e_buf[pl.ds(base, L)] = ev
            idx = base + lanes
            valid = (idx >= ra) & (idx < rb)
            return jnp.maximum(mv, jnp.where(valid, ev, NEG_BIG))

        mv = lax.fori_loop(g_lo, g_hi, grp_a,
                           jnp.full((L,), NEG_BIG, jnp.float32))
        m_old = m_buf[j, :]
        m_new = jnp.maximum(m_old, allmax(mv))
        scale = jnp.exp(m_old - m_new)
        m_buf[j, :] = m_new

        r_vecs = []
        for k in range(KD):
            r_vecs.append(r_buf[j, pl.ds(k * L, L)] * scale)

        def grp_b(gi, carry):
            lv = carry[0]
            rs = list(carry[1:])
            base = gi * L
            ev = e_buf[pl.ds(base, L)]
            idx = base + lanes
            valid = (idx >= ra) & (idx < rb)
            wv = jnp.exp(jnp.where(valid, ev - m_new, NEG_BIG))
            lv = lv + wv
            for p in range(L):
                wp = wv.at[jnp.full((L,), p, jnp.int32)].get(
                    mode="promise_in_bounds")
                for k in range(KD):
                    rs[k] = rs[k] + wp * buf[base + p, pl.ds(k * L, L)]
            return (lv,) + tuple(rs)

        out = lax.fori_loop(g_lo, g_hi, grp_b, (zero_v,) + tuple(r_vecs))
        for k in range(KD):
            r_buf[j, pl.ds(k * L, L)] = out[1 + k]
        l_buf[j, :] = l_buf[j, :] * scale + allsum(out[0])

    def process(ci, slot):
        cs = chunk_start(ci)
        lo = base0 + ci * CHUNK
        hi = jnp.minimum(lo + CHUNK, w_hi)

        def seg_body(j, carry):
            a = jnp.maximum(off_s[j], lo)
            b = jnp.minimum(off_s[j + 1], hi)

            @pl.when(b > a)
            def _():
                process_span(j, chunks.at[slot], a - cs, b - cs)
            return carry
        lax.fori_loop(0, SEG_PER_W, seg_body, 0)

    for slot in range(2):
        @pl.when(slot < n_chunks)
        def _():
            issue(slot, slot, sems[slot])

    def outer(pi, carry):
        for slot in range(2):
            ci = pi * 2 + slot

            @pl.when(ci < n_chunks)
            def _():
                wait(slot, sems[slot])
                process(ci, slot)

                @pl.when(ci + 2 < n_chunks)
                def _():
                    issue(ci + 2, slot, sems[slot])
        return carry

    n_pairs = (n_chunks + 1) // 2
    lax.fori_loop(0, n_pairs, outer, 0)

    for j in range(SEG_PER_W):
        l_vec = l_buf[j, :]
        inv = jnp.where(l_vec > 0.0, 1.0 / l_vec, jnp.zeros((L,), jnp.float32))
        for k in range(KD):
            r_buf[j, pl.ds(k * L, L)] = r_buf[j, pl.ds(k * L, L)] * inv
    pltpu.sync_copy(r_buf, out_hbm.at[pl.ds(seg0, SEG_PER_W)])


_sc_readout = functools.partial(
    pl.kernel,
    out_type=jax.ShapeDtypeStruct((B, D), jnp.float32),
    mesh=plsc.VectorSubcoreMesh(core_axis_name="c", subcore_axis_name="s"),
    scratch_types=[
        pltpu.VMEM((L,), jnp.int32),
        pltpu.SMEM((L,), jnp.int32),
        pltpu.VMEM((SEG_PER_W, D), jnp.float32),
        pltpu.VMEM((2, CHUNK, D), jnp.float32),
        pltpu.VMEM((CHUNK,), jnp.float32),
        pltpu.VMEM((SEG_PER_W, D), jnp.float32),
        pltpu.VMEM((SEG_PER_W, L), jnp.float32),
        pltpu.VMEM((SEG_PER_W, L), jnp.float32),
        pltpu.SemaphoreType.DMA,
        pltpu.SemaphoreType.DMA,
    ],
)(_sc_readout_body)


def kernel(feat, W_ih, W_hh, b_ih, b_hh, segment_ids):
    feat = feat.astype(jnp.float32)
    seg = segment_ids.astype(jnp.int32)
    offs = jnp.searchsorted(seg, jnp.arange(B + L, dtype=jnp.int32),
                            side="left").astype(jnp.int32)
    bih = b_ih.reshape(1, 4 * D).astype(jnp.float32)
    bhh = b_hh.reshape(1, 4 * D).astype(jnp.float32)

    h = jnp.zeros((B, D), jnp.float32)
    c = jnp.zeros((B, D), jnp.float32)
    q_star = jnp.zeros((B, 2 * D), jnp.float32)
    for _ in range(N_ITERS):
        h, c = _lstm_step(q_star, h, c, W_ih, W_hh, bih, bhh)
        readout = _sc_readout(feat, h, offs)
        q_star = jnp.concatenate([h, readout], axis=-1)
    return q_star


# speedup vs baseline: 1.0100x; 1.0100x over previous
import functools

import jax
import jax.numpy as jnp
from jax import lax
from jax.experimental import pallas as pl
from jax.experimental.pallas import tpu as pltpu
from jax.experimental.pallas import tpu_sc as plsc

N = 100000
D = 128
B = 256
N_ITERS = 3

L = 16
KD = D // L
NC = 2
NS = 16
NW = NC * NS
SEG_PER_W = B // NW
CHUNK = 256
NEG_BIG = -3.4e38


def _lstm_body(hp_ref, rp_ref, h_ref, c_ref, wih_ref, whh_ref, bih_ref,
               bhh_ref, h_out, c_out):
    h = h_ref[...]
    gates = lax.dot_general(hp_ref[...], wih_ref[:, :D],
                            (((1,), (1,)), ((), ())),
                            preferred_element_type=jnp.float32)
    gates = gates + lax.dot_general(rp_ref[...], wih_ref[:, D:],
                                    (((1,), (1,)), ((), ())),
                                    preferred_element_type=jnp.float32)
    gates = gates + lax.dot_general(h, whh_ref[...], (((1,), (1,)), ((), ())),
                                    preferred_element_type=jnp.float32)
    gates = gates + bih_ref[...] + bhh_ref[...]
    i_ = jax.nn.sigmoid(gates[:, 0 * D:1 * D])
    f_ = jax.nn.sigmoid(gates[:, 1 * D:2 * D])
    g_ = jnp.tanh(gates[:, 2 * D:3 * D])
    o_ = jax.nn.sigmoid(gates[:, 3 * D:4 * D])
    c_new = f_ * c_ref[...] + i_ * g_
    h_out[...] = o_ * jnp.tanh(c_new)
    c_out[...] = c_new


def _lstm_step(h_prev, r_prev, h, c, W_ih, W_hh, b_ih, b_hh):
    return pl.pallas_call(
        _lstm_body,
        out_shape=(jax.ShapeDtypeStruct((B, D), jnp.float32),
                   jax.ShapeDtypeStruct((B, D), jnp.float32)),
    )(h_prev, r_prev, h, c, W_ih, W_hh, b_ih, b_hh)


def _sc_readout_body(feat_hbm, q_hbm, offs_hbm, out_hbm,
                     off_v, off_s, q_loc, chunks, e_buf,
                     r_buf, m_buf, l_buf, sem0, sem1):
    wid = lax.axis_index("s") * NC + lax.axis_index("c")
    seg0 = wid * SEG_PER_W
    lanes = lax.iota(jnp.int32, 16)
    xor_idx = [jnp.bitwise_xor(lanes, s) for s in (1, 2, 4, 8)]

    def allsum(v):
        for idx in xor_idx:
            v = v + v.at[idx].get(mode="promise_in_bounds")
        return v

    def allmax(v):
        for idx in xor_idx:
            v = jnp.maximum(v, v.at[idx].get(mode="promise_in_bounds"))
        return v

    pltpu.sync_copy(offs_hbm.at[pl.ds(seg0, L)], off_v)
    pltpu.sync_copy(q_hbm.at[pl.ds(seg0, SEG_PER_W)], q_loc)
    offv = off_v[...]
    for j in range(SEG_PER_W + 1):
        off_s[j] = offv[j]

    zero_v = jnp.zeros((L,), jnp.float32)
    for j in range(SEG_PER_W):
        m_buf[j, :] = jnp.full((L,), NEG_BIG, jnp.float32)
        l_buf[j, :] = zero_v
        for k in range(KD):
            r_buf[j, pl.ds(k * L, L)] = zero_v

    w_lo = offv[0]
    w_hi = offv[SEG_PER_W]
    base0 = (w_lo // 8) * 8
    n_chunks = (w_hi - base0 + CHUNK - 1) // CHUNK
    sems = [sem0, sem1]

    def chunk_start(ci):
        return jnp.minimum(base0 + ci * CHUNK, N - CHUNK)

    def issue(ci, slot, sem):
        pltpu.make_async_copy(
            feat_hbm.at[pl.ds(chunk_start(ci), CHUNK)],
            chunks.at[slot], sem).start()

    def wait(slot, sem):
        pltpu.make_async_copy(
            feat_hbm.at[pl.ds(0, CHUNK)], chunks.at[slot], sem).wait()

    def process_span(j, buf, ra, rb):
        q_vecs = [q_loc[j, pl.ds(k * L, L)] for k in range(KD)]
        g_lo = ra // L
        g_hi = (rb + L - 1) // L

        def grp_a(gi, mv):
            base = gi * L
            ev = zero_v
            for p in range(L):
                acc = buf[base + p, pl.ds(0, L)] * q_vecs[0]
                for k in range(1, KD):
                    acc = acc + buf[base + p, pl.ds(k * L, L)] * q_vecs[k]
                ev = jnp.where(lanes == p, allsum(acc), ev)
            e_buf[pl.ds(base, L)] = ev
            idx = base + lanes
            valid = (idx >= ra) & (idx < rb)
            return jnp.maximum(mv, jnp.where(valid, ev, NEG_BIG))

        mv = lax.fori_loop(g_lo, g_hi, grp_a,
                           jnp.full((L,), NEG_BIG, jnp.float32))
        m_old = m_buf[j, :]
        m_new = jnp.maximum(m_old, allmax(mv))
        scale = jnp.exp(m_old - m_new)
        m_buf[j, :] = m_new

        r_vecs = []
        for k in range(KD):
            r_vecs.append(r_buf[j, pl.ds(k * L, L)] * scale)

        def grp_b(gi, carry):
            lv = carry[0]
            rs = list(carry[1:])
            base = gi * L
            ev = e_buf[pl.ds(base, L)]
            idx = base + lanes
            valid = (idx >= ra) & (idx < rb)
            wv = jnp.exp(jnp.where(valid, ev - m_new, NEG_BIG))
            lv = lv + wv
            for p in range(L):
                wp = wv.at[jnp.full((L,), p, jnp.int32)].get(
                    mode="promise_in_bounds")
                for k in range(KD):
                    rs[k] = rs[k] + wp * buf[base + p, pl.ds(k * L, L)]
            return (lv,) + tuple(rs)

        out = lax.fori_loop(g_lo, g_hi, grp_b, (zero_v,) + tuple(r_vecs))
        for k in range(KD):
            r_buf[j, pl.ds(k * L, L)] = out[1 + k]
        l_buf[j, :] = l_buf[j, :] * scale + allsum(out[0])

    def process(ci, slot):
        cs = chunk_start(ci)
        lo = base0 + ci * CHUNK
        hi = jnp.minimum(lo + CHUNK, w_hi)

        def seg_body(j, carry):
            a = jnp.maximum(off_s[j], lo)
            b = jnp.minimum(off_s[j + 1], hi)

            @pl.when(b > a)
            def _():
                process_span(j, chunks.at[slot], a - cs, b - cs)
            return carry
        lax.fori_loop(0, SEG_PER_W, seg_body, 0)

    for slot in range(2):
        @pl.when(slot < n_chunks)
        def _():
            issue(slot, slot, sems[slot])

    def outer(pi, carry):
        for slot in range(2):
            ci = pi * 2 + slot

            @pl.when(ci < n_chunks)
            def _():
                wait(slot, sems[slot])
                process(ci, slot)

                @pl.when(ci + 2 < n_chunks)
                def _():
                    issue(ci + 2, slot, sems[slot])
        return carry

    n_pairs = (n_chunks + 1) // 2
    lax.fori_loop(0, n_pairs, outer, 0)

    for j in range(SEG_PER_W):
        l_vec = l_buf[j, :]
        inv = jnp.where(l_vec > 0.0, 1.0 / l_vec, jnp.zeros((L,), jnp.float32))
        for k in range(KD):
            r_buf[j, pl.ds(k * L, L)] = r_buf[j, pl.ds(k * L, L)] * inv
    pltpu.sync_copy(r_buf, out_hbm.at[pl.ds(seg0, SEG_PER_W)])


_sc_readout = functools.partial(
    pl.kernel,
    out_type=jax.ShapeDtypeStruct((B, D), jnp.float32),
    mesh=plsc.VectorSubcoreMesh(core_axis_name="c", subcore_axis_name="s"),
    scratch_types=[
        pltpu.VMEM((L,), jnp.int32),
        pltpu.SMEM((L,), jnp.int32),
        pltpu.VMEM((SEG_PER_W, D), jnp.float32),
        pltpu.VMEM((2, CHUNK, D), jnp.float32),
        pltpu.VMEM((CHUNK,), jnp.float32),
        pltpu.VMEM((SEG_PER_W, D), jnp.float32),
        pltpu.VMEM((SEG_PER_W, L), jnp.float32),
        pltpu.VMEM((SEG_PER_W, L), jnp.float32),
        pltpu.SemaphoreType.DMA,
        pltpu.SemaphoreType.DMA,
    ],
)(_sc_readout_body)


def kernel(feat, W_ih, W_hh, b_ih, b_hh, segment_ids):
    feat = feat.astype(jnp.float32)
    seg = segment_ids.astype(jnp.int32)
    offs = jnp.searchsorted(seg, jnp.arange(B + L, dtype=jnp.int32),
                            side="left").astype(jnp.int32)
    bih = b_ih.reshape(1, 4 * D).astype(jnp.float32)
    bhh = b_hh.reshape(1, 4 * D).astype(jnp.float32)

    h = jnp.zeros((B, D), jnp.float32)
    c = jnp.zeros((B, D), jnp.float32)
    readout = jnp.zeros((B, D), jnp.float32)
    for _ in range(N_ITERS):
        h, c = _lstm_step(h, readout, h, c, W_ih, W_hh, bih, bhh)
        readout = _sc_readout(feat, h, offs)
    return jnp.concatenate([h, readout], axis=-1)
